# baseline (speedup 1.0000x reference)
import jax
import jax.numpy as jnp
from jax.experimental import pallas as pl
from jax.experimental.pallas import tpu as pltpu

_B = 128
_EMB = 768
_POOL = 100
_PLEN = 8
_HALF = _PLEN // 2
_EPS = 1e-6


def _body(gate_ref, x_ref, ea_ref, ek_ref, ep_ref, xb_ref,
          eko_ref, evo_ref, xbo_ref, sem):
    cp = pltpu.make_async_copy(xb_ref, xbo_ref, sem)
    cp.start()

    ea = ea_ref[...]
    m = jnp.max(ea, axis=1, keepdims=True)
    p = jnp.exp(ea - m)
    A = p / jnp.sum(p, axis=1, keepdims=True)

    ek = ek_ref[...]
    n2 = jnp.sqrt(jnp.sum(ek * ek, axis=1, keepdims=True))
    Wn = (A * ek) / jnp.maximum(n2, _EPS)

    x = x_ref[...]
    dn_t = (((1,), (1,)), ((), ()))
    num = jax.lax.dot_general(x, Wn, dn_t, preferred_element_type=jnp.float32)
    n1sq = jax.lax.dot_general(x * x, A * A, dn_t,
                               preferred_element_type=jnp.float32)
    n1 = jnp.maximum(jnp.sqrt(n1sq), _EPS)

    gate = gate_ref[0]
    aq = ((num / n1) + 1.0) * (0.5 * gate)

    dn = (((1,), (0,)), ((), ()))
    for l in range(_PLEN):
        dst = eko_ref if l < _HALF else evo_ref
        j = l if l < _HALF else l - _HALF
        dst[:, j * _EMB:(j + 1) * _EMB] = jax.lax.dot_general(
            aq, ep_ref[l], dn, preferred_element_type=jnp.float32)

    cp.wait()


def kernel(x_querry, x_block, e_p_0, e_k_0, e_a_0, l):
    in_layers = jnp.any(jnp.asarray(l) == jnp.asarray([0, 1, 2, 3, 4, 5]))
    gate = in_layers.astype(jnp.float32).reshape(1)

    out_t = (
        jax.ShapeDtypeStruct((_B, _HALF * _EMB), jnp.float32),
        jax.ShapeDtypeStruct((_B, _HALF * _EMB), jnp.float32),
        jax.ShapeDtypeStruct(x_block.shape, x_block.dtype),
    )
    ek2, ev2, xb_out = pl.pallas_call(
        _body,
        out_shape=out_t,
        in_specs=[
            pl.BlockSpec(memory_space=pltpu.SMEM),
            pl.BlockSpec(memory_space=pltpu.VMEM),
            pl.BlockSpec(memory_space=pltpu.VMEM),
            pl.BlockSpec(memory_space=pltpu.VMEM),
            pl.BlockSpec(memory_space=pltpu.VMEM),
            pl.BlockSpec(memory_space=pltpu.MemorySpace.HBM),
        ],
        out_specs=(
            pl.BlockSpec(memory_space=pltpu.VMEM),
            pl.BlockSpec(memory_space=pltpu.VMEM),
            pl.BlockSpec(memory_space=pltpu.MemorySpace.HBM),
        ),
        scratch_shapes=[pltpu.SemaphoreType.DMA],
    )(gate, x_querry, e_a_0, e_k_0, e_p_0, x_block)

    Ek = ek2.reshape(_B, _HALF, _EMB)
    Ev = ev2.reshape(_B, _HALF, _EMB)
    return (Ek, Ev, xb_out)


# speedup vs baseline: 7.0170x; 7.0170x over previous
import jax
import jax.numpy as jnp
from jax.experimental import pallas as pl
from jax.experimental.pallas import tpu as pltpu

_B = 128
_SEQ = 197
_EMB = 768
_POOL = 100
_PLEN = 8
_HALF = _PLEN // 2
_EPS = 1e-6

_ROWS = _B * _SEQ
_CHUNKS = 16
_CROWS = _ROWS // _CHUNKS


def _body(gate_ref, x_ref, ea_ref, ek_ref, ep_ref, xb_ref,
          eko_ref, evo_ref, xbo_ref):
    xbo_ref[...] = xb_ref[...]

    @pl.when(pl.program_id(0) == 0)
    def _compute():
        ea = ea_ref[...]
        m = jnp.max(ea, axis=1, keepdims=True)
        p = jnp.exp(ea - m)
        A = p / jnp.sum(p, axis=1, keepdims=True)

        ek = ek_ref[...]
        n2 = jnp.sqrt(jnp.sum(ek * ek, axis=1, keepdims=True))
        Wn = (A * ek) / jnp.maximum(n2, _EPS)

        x = x_ref[...]
        dn_t = (((1,), (1,)), ((), ()))
        num = jax.lax.dot_general(x, Wn, dn_t,
                                  preferred_element_type=jnp.float32)
        n1sq = jax.lax.dot_general(x * x, A * A, dn_t,
                                   preferred_element_type=jnp.float32)
        n1 = jnp.maximum(jnp.sqrt(n1sq), _EPS)

        gate = gate_ref[0]
        aq = ((num / n1) + 1.0) * (0.5 * gate)

        dn = (((1,), (0,)), ((), ()))
        for l in range(_PLEN):
            dst = eko_ref if l < _HALF else evo_ref
            j = l if l < _HALF else l - _HALF
            dst[:, j * _EMB:(j + 1) * _EMB] = jax.lax.dot_general(
                aq, ep_ref[l], dn, preferred_element_type=jnp.float32)


def kernel(x_querry, x_block, e_p_0, e_k_0, e_a_0, l):
    in_layers = jnp.any(jnp.asarray(l) == jnp.asarray([0, 1, 2, 3, 4, 5]))
    gate = in_layers.astype(jnp.float32).reshape(1)

    xb2 = x_block.reshape(_ROWS, _EMB)

    out_t = (
        jax.ShapeDtypeStruct((_B, _HALF * _EMB), jnp.float32),
        jax.ShapeDtypeStruct((_B, _HALF * _EMB), jnp.float32),
        jax.ShapeDtypeStruct((_ROWS, _EMB), jnp.float32),
    )
    full = lambda i: (0, 0)
    full3 = lambda i: (0, 0, 0)
    ek2, ev2, xb_out = pl.pallas_call(
        _body,
        grid=(_CHUNKS,),
        out_shape=out_t,
        in_specs=[
            pl.BlockSpec(memory_space=pltpu.SMEM),
            pl.BlockSpec((_B, _EMB), full),
            pl.BlockSpec((_POOL, _EMB), full),
            pl.BlockSpec((_POOL, _EMB), full),
            pl.BlockSpec((_PLEN, _POOL, _EMB), full3),
            pl.BlockSpec((_CROWS, _EMB), lambda i: (i, 0)),
        ],
        out_specs=(
            pl.BlockSpec((_B, _HALF * _EMB), full),
            pl.BlockSpec((_B, _HALF * _EMB), full),
            pl.BlockSpec((_CROWS, _EMB), lambda i: (i, 0)),
        ),
    )(gate, x_querry, e_a_0, e_k_0, e_p_0, xb2)

    Ek = ek2.reshape(_B, _HALF, _EMB)
    Ev = ev2.reshape(_B, _HALF, _EMB)
    return (Ek, Ev, xb_out.reshape(_B, _SEQ, _EMB))


# speedup vs baseline: 37.3144x; 5.3177x over previous
import jax
import jax.numpy as jnp
from jax.experimental import pallas as pl
from jax.experimental.pallas import tpu as pltpu

_B = 128
_EMB = 768
_POOL = 100
_PLEN = 8
_HALF = _PLEN // 2
_EPS = 1e-6


def _body(gate_ref, x_ref, ea_ref, ek_ref, ep_ref, eko_ref, evo_ref, aq_ref):
    l = pl.program_id(0)

    @pl.when(l == 0)
    def _scores():
        ea = ea_ref[...]
        m = jnp.max(ea, axis=1, keepdims=True)
        p = jnp.exp(ea - m)
        A = p / jnp.sum(p, axis=1, keepdims=True)

        ek = ek_ref[...]
        n2 = jnp.sqrt(jnp.sum(ek * ek, axis=1, keepdims=True))
        Wn = (A * ek) / jnp.maximum(n2, _EPS)

        x = x_ref[...]
        dn_t = (((1,), (1,)), ((), ()))
        num = jax.lax.dot_general(x, Wn, dn_t,
                                  preferred_element_type=jnp.float32)
        n1sq = jax.lax.dot_general(x * x, A * A, dn_t,
                                   preferred_element_type=jnp.float32)
        n1 = jnp.maximum(jnp.sqrt(n1sq), _EPS)
        aq_ref[...] = ((num / n1) + 1.0) * (0.5 * gate_ref[0])

    pslice = jax.lax.dot_general(
        aq_ref[...], ep_ref[0], (((1,), (0,)), ((), ())),
        preferred_element_type=jnp.float32)

    @pl.when(l < _HALF)
    def _wk():
        eko_ref[...] = pslice

    @pl.when(l >= _HALF)
    def _wv():
        evo_ref[...] = pslice


def kernel(x_querry, x_block, e_p_0, e_k_0, e_a_0, l):
    in_layers = jnp.any(jnp.asarray(l) == jnp.asarray([0, 1, 2, 3, 4, 5]))
    gate = in_layers.astype(jnp.float32).reshape(1)

    out_t = (
        jax.ShapeDtypeStruct((_B, _HALF * _EMB), jnp.float32),
        jax.ShapeDtypeStruct((_B, _HALF * _EMB), jnp.float32),
    )
    full = lambda i: (0, 0)
    ek2, ev2 = pl.pallas_call(
        _body,
        grid=(_PLEN,),
        out_shape=out_t,
        in_specs=[
            pl.BlockSpec(memory_space=pltpu.SMEM),
            pl.BlockSpec((_B, _EMB), full),
            pl.BlockSpec((_POOL, _EMB), full),
            pl.BlockSpec((_POOL, _EMB), full),
            pl.BlockSpec((1, _POOL, _EMB), lambda i: (i, 0, 0)),
        ],
        out_specs=(
            pl.BlockSpec((_B, _EMB), lambda i: (0, jnp.minimum(i, _HALF - 1))),
            pl.BlockSpec((_B, _EMB), lambda i: (0, jnp.maximum(i - _HALF, 0))),
        ),
        scratch_shapes=[pltpu.VMEM((_B, _POOL), jnp.float32)],
    )(gate, x_querry, e_a_0, e_k_0, e_p_0)

    Ek = ek2.reshape(_B, _HALF, _EMB)
    Ev = ev2.reshape(_B, _HALF, _EMB)
    return (Ek, Ev, x_block)


# speedup vs baseline: 37.8596x; 1.0146x over previous
import jax
import jax.numpy as jnp
from jax.experimental import pallas as pl
from jax.experimental.pallas import tpu as pltpu

_B = 128
_EMB = 768
_POOL = 100
_PLEN = 8
_HALF = _PLEN // 2
_EPS = 1e-6


def _body(gate_ref, x_ref, w_ref, eko_ref, evo_ref):
    ea = w_ref[0:_POOL, :]
    m = jnp.max(ea, axis=1, keepdims=True)
    p = jnp.exp(ea - m)
    A = p / jnp.sum(p, axis=1, keepdims=True)

    ek = w_ref[_POOL:2 * _POOL, :]
    n2 = jnp.sqrt(jnp.sum(ek * ek, axis=1, keepdims=True))
    Wn = (A * ek) / jnp.maximum(n2, _EPS)

    x = x_ref[...]
    dn_t = (((1,), (1,)), ((), ()))
    num = jax.lax.dot_general(x, Wn, dn_t,
                              preferred_element_type=jnp.float32)
    n1sq = jax.lax.dot_general(x * x, A * A, dn_t,
                               preferred_element_type=jnp.float32)
    n1 = jnp.maximum(jnp.sqrt(n1sq), _EPS)
    aq = ((num / n1) + 1.0) * (0.5 * gate_ref[0])

    dn = (((1,), (0,)), ((), ()))
    for l in range(_PLEN):
        dst = eko_ref if l < _HALF else evo_ref
        j = l if l < _HALF else l - _HALF
        epl = w_ref[(2 + l) * _POOL:(3 + l) * _POOL, :]
        dst[:, j * _EMB:(j + 1) * _EMB] = jax.lax.dot_general(
            aq, epl, dn, preferred_element_type=jnp.float32)


def kernel(x_querry, x_block, e_p_0, e_k_0, e_a_0, l):
    in_layers = jnp.any(jnp.asarray(l) == jnp.asarray([0, 1, 2, 3, 4, 5]))
    gate = in_layers.astype(jnp.float32).reshape(1)

    w = jnp.concatenate(
        [e_a_0, e_k_0, e_p_0.reshape(_PLEN * _POOL, _EMB)], axis=0)

    out_t = (
        jax.ShapeDtypeStruct((_B, _HALF * _EMB), jnp.float32),
        jax.ShapeDtypeStruct((_B, _HALF * _EMB), jnp.float32),
    )
    ek2, ev2 = pl.pallas_call(
        _body,
        out_shape=out_t,
        in_specs=[
            pl.BlockSpec(memory_space=pltpu.SMEM),
            pl.BlockSpec(memory_space=pltpu.VMEM),
            pl.BlockSpec(memory_space=pltpu.VMEM),
        ],
        out_specs=(
            pl.BlockSpec(memory_space=pltpu.VMEM),
            pl.BlockSpec(memory_space=pltpu.VMEM),
        ),
    )(gate, x_querry, w)

    Ek = ek2.reshape(_B, _HALF, _EMB)
    Ev = ev2.reshape(_B, _HALF, _EMB)
    return (Ek, Ev, x_block)


# speedup vs baseline: 40.0503x; 1.0579x over previous
import jax
import jax.numpy as jnp
from jax.experimental import pallas as pl
from jax.experimental.pallas import tpu as pltpu

_B = 128
_EMB = 768
_POOL = 100
_PLEN = 8
_HALF = _PLEN // 2
_EPS = 1e-6

_GATED_LAYERS = (0, 1, 2, 3, 4, 5)


def _body(l_ref, x_ref, ea_ref, ek_ref, ep_ref, eko_ref, evo_ref):
    lv = l_ref[0]
    gate = jnp.where(
        (lv >= _GATED_LAYERS[0]) & (lv <= _GATED_LAYERS[-1]), 1.0, 0.0
    ).astype(jnp.float32)

    ea = ea_ref[...]
    m = jnp.max(ea, axis=1, keepdims=True)
    p = jnp.exp(ea - m)
    A = p / jnp.sum(p, axis=1, keepdims=True)

    ek = ek_ref[...]
    n2 = jnp.sqrt(jnp.sum(ek * ek, axis=1, keepdims=True))
    Wn = (A * ek) / jnp.maximum(n2, _EPS)

    x = x_ref[...]
    dn_t = (((1,), (1,)), ((), ()))
    num = jax.lax.dot_general(x, Wn, dn_t,
                              preferred_element_type=jnp.float32)
    n1sq = jax.lax.dot_general(x * x, A * A, dn_t,
                               preferred_element_type=jnp.float32)
    n1 = jnp.maximum(jnp.sqrt(n1sq), _EPS)
    aq = ((num / n1) + 1.0) * (0.5 * gate)

    dn = (((1,), (0,)), ((), ()))
    for l in range(_PLEN):
        dst = eko_ref if l < _HALF else evo_ref
        j = l if l < _HALF else l - _HALF
        dst[:, j * _EMB:(j + 1) * _EMB] = jax.lax.dot_general(
            aq, ep_ref[l], dn, preferred_element_type=jnp.float32)


def kernel(x_querry, x_block, e_p_0, e_k_0, e_a_0, l):
    li = jnp.asarray(l, jnp.int32).reshape(1)

    out_t = (
        jax.ShapeDtypeStruct((_B, _HALF * _EMB), jnp.float32),
        jax.ShapeDtypeStruct((_B, _HALF * _EMB), jnp.float32),
    )
    ek2, ev2 = pl.pallas_call(
        _body,
        out_shape=out_t,
        in_specs=[
            pl.BlockSpec(memory_space=pltpu.SMEM),
            pl.BlockSpec(memory_space=pltpu.VMEM),
            pl.BlockSpec(memory_space=pltpu.VMEM),
            pl.BlockSpec(memory_space=pltpu.VMEM),
            pl.BlockSpec(memory_space=pltpu.VMEM),
        ],
        out_specs=(
            pl.BlockSpec(memory_space=pltpu.VMEM),
            pl.BlockSpec(memory_space=pltpu.VMEM),
        ),
    )(li, x_querry, e_a_0, e_k_0, e_p_0)

    Ek = ek2.reshape(_B, _HALF, _EMB)
    Ev = ev2.reshape(_B, _HALF, _EMB)
    return (Ek, Ev, x_block)


# speedup vs baseline: 40.0703x; 1.0005x over previous
import jax
import jax.numpy as jnp
from jax.experimental import pallas as pl
from jax.experimental.pallas import tpu as pltpu

_B = 128
_EMB = 768
_POOL = 100
_PLEN = 8
_HALF = _PLEN // 2
_EPS = 1e-6

_GATED_LAYERS = (0, 1, 2, 3, 4, 5)


def _body(l_ref, x_ref, ea_ref, ek_ref, ep_ref, eko_ref, evo_ref):
    lv = l_ref[...]
    gate = jnp.where(
        (lv >= _GATED_LAYERS[0]) & (lv <= _GATED_LAYERS[-1]), 1.0, 0.0
    ).astype(jnp.float32)

    ea = ea_ref[...]
    m = jnp.max(ea, axis=1, keepdims=True)
    p = jnp.exp(ea - m)
    A = p / jnp.sum(p, axis=1, keepdims=True)

    ek = ek_ref[...]
    n2 = jnp.sqrt(jnp.sum(ek * ek, axis=1, keepdims=True))
    Wn = (A * ek) / jnp.maximum(n2, _EPS)

    x = x_ref[...]
    dn_t = (((1,), (1,)), ((), ()))
    num = jax.lax.dot_general(x, Wn, dn_t,
                              preferred_element_type=jnp.float32)
    n1sq = jax.lax.dot_general(x * x, A * A, dn_t,
                               preferred_element_type=jnp.float32)
    n1 = jnp.maximum(jnp.sqrt(n1sq), _EPS)
    aq = ((num / n1) + 1.0) * (0.5 * gate)

    dn = (((1,), (0,)), ((), ()))
    for l in range(_PLEN):
        dst = eko_ref if l < _HALF else evo_ref
        j = l if l < _HALF else l - _HALF
        dst[:, j * _EMB:(j + 1) * _EMB] = jax.lax.dot_general(
            aq, ep_ref[l], dn, preferred_element_type=jnp.float32)


def kernel(x_querry, x_block, e_p_0, e_k_0, e_a_0, l):
    li = jnp.asarray(l, jnp.int32)

    out_t = (
        jax.ShapeDtypeStruct((_B, _HALF * _EMB), jnp.float32),
        jax.ShapeDtypeStruct((_B, _HALF * _EMB), jnp.float32),
    )
    ek2, ev2 = pl.pallas_call(
        _body,
        out_shape=out_t,
        in_specs=[
            pl.BlockSpec(memory_space=pltpu.SMEM),
            pl.BlockSpec(memory_space=pltpu.VMEM),
            pl.BlockSpec(memory_space=pltpu.VMEM),
            pl.BlockSpec(memory_space=pltpu.VMEM),
            pl.BlockSpec(memory_space=pltpu.VMEM),
        ],
        out_specs=(
            pl.BlockSpec(memory_space=pltpu.VMEM),
            pl.BlockSpec(memory_space=pltpu.VMEM),
        ),
    )(li, x_querry, e_a_0, e_k_0, e_p_0)

    Ek = ek2.reshape(_B, _HALF, _EMB)
    Ev = ev2.reshape(_B, _HALF, _EMB)
    return (Ek, Ev, x_block)
